# baseline (speedup 1.0000x reference)
import functools

import jax
import jax.numpy as jnp
from jax import lax
from jax.experimental import pallas as pl
from jax.experimental.pallas import tpu as pltpu
from jax.experimental.pallas import tpu_sc as plsc

_B = 16384
_L = 16
_NC = 1
_NS = 16
_NW = _NC * _NS
_BPW = _B // _NW
_NT = 100
_NCH = 4
_CH = _BPW // _NCH

_mesh = plsc.VectorSubcoreMesh(
    core_axis_name="c", subcore_axis_name="s", num_cores=_NC)


@functools.partial(
    pl.kernel,
    mesh=_mesh,
    out_type=jax.ShapeDtypeStruct((_B,), jnp.float32),
    scratch_types=[
        pltpu.VMEM((_BPW,), jnp.int32),
        pltpu.VMEM((_NT,), jnp.float32),
        pltpu.VMEM((_BPW,), jnp.float32),
        pltpu.SemaphoreType.DMA,
        [pltpu.SemaphoreType.DMA] * _NCH,
        pltpu.SemaphoreType.DMA,
    ],
    compiler_params=pltpu.CompilerParams(needs_layout_passes=False),
)
def _gather_kernel(tasks_hbm, table_hbm, out_hbm, idx_v, tab_v, out_v,
                   sem_t, sems_i, sem_o):
    wid = lax.axis_index("s") * _NC + lax.axis_index("c")
    base = wid * _BPW
    cp_t = pltpu.async_copy(table_hbm, tab_v, sem_t)
    cps_i = [
        pltpu.async_copy(
            tasks_hbm.at[pl.ds(base + c * _CH, _CH)],
            idx_v.at[pl.ds(c * _CH, _CH)],
            sems_i[c],
        )
        for c in range(_NCH)
    ]
    cp_t.wait()
    cps_o = []
    for c in range(_NCH):
        cps_i[c].wait()
        for j in range(c * (_CH // _L), (c + 1) * (_CH // _L)):
            idx16 = idx_v[pl.ds(j * _L, _L)]
            out_v[pl.ds(j * _L, _L)] = plsc.load_gather(tab_v, [idx16])
        cps_o.append(
            pltpu.async_copy(
                out_v.at[pl.ds(c * _CH, _CH)],
                out_hbm.at[pl.ds(base + c * _CH, _CH)],
                sem_o,
            )
        )
    for cp in cps_o:
        cp.wait()


def kernel(tasks, table):
    out = _gather_kernel(tasks, table.reshape(-1))
    return out.reshape(_B, 1, 1, 1)


# speedup vs baseline: 1.0114x; 1.0114x over previous
import functools

import jax
import jax.numpy as jnp
from jax import lax
from jax.experimental import pallas as pl
from jax.experimental.pallas import tpu as pltpu
from jax.experimental.pallas import tpu_sc as plsc

_B = 16384
_L = 16
_NC = 1
_NS = 16
_NW = _NC * _NS
_BPW = _B // _NW
_NT = 100

_mesh = plsc.VectorSubcoreMesh(core_axis_name="c", subcore_axis_name="s", num_cores=1)


@functools.partial(
    pl.kernel,
    mesh=_mesh,
    out_type=jax.ShapeDtypeStruct((_B,), jnp.float32),
    scratch_types=[
        pltpu.VMEM((_BPW,), jnp.int32),
        pltpu.VMEM((_NT,), jnp.float32),
        pltpu.VMEM((_BPW,), jnp.float32),
        pltpu.SemaphoreType.DMA,
        pltpu.SemaphoreType.DMA,
        pltpu.SemaphoreType.DMA,
        pltpu.SemaphoreType.DMA,
    ],
    compiler_params=pltpu.CompilerParams(needs_layout_passes=False),
)
def _gather_kernel(tasks_hbm, table_hbm, out_hbm, idx_v, tab_v, out_v,
                   sem_i0, sem_i1, sem_t, sem_o):
    wid = lax.axis_index("s") * _NC + lax.axis_index("c")
    base = wid * _BPW
    half = _BPW // 2
    cp_t = pltpu.async_copy(table_hbm, tab_v, sem_t)
    cp_i0 = pltpu.async_copy(
        tasks_hbm.at[pl.ds(base, half)], idx_v.at[pl.ds(0, half)], sem_i0)
    cp_i1 = pltpu.async_copy(
        tasks_hbm.at[pl.ds(base + half, half)], idx_v.at[pl.ds(half, half)], sem_i1)
    cp_t.wait()
    cp_i0.wait()
    for j in range(half // _L):
        idx16 = idx_v[pl.ds(j * _L, _L)]
        out_v[pl.ds(j * _L, _L)] = plsc.load_gather(tab_v, [idx16])
    cp_o0 = pltpu.async_copy(
        out_v.at[pl.ds(0, half)], out_hbm.at[pl.ds(base, half)], sem_o)
    cp_i1.wait()
    for j in range(half // _L, _BPW // _L):
        idx16 = idx_v[pl.ds(j * _L, _L)]
        out_v[pl.ds(j * _L, _L)] = plsc.load_gather(tab_v, [idx16])
    cp_o1 = pltpu.async_copy(
        out_v.at[pl.ds(half, half)], out_hbm.at[pl.ds(base + half, half)], sem_o)
    cp_o0.wait()
    cp_o1.wait()


def kernel(tasks, table):
    out = _gather_kernel(tasks, table.reshape(-1))
    return out.reshape(_B, 1, 1, 1)
